# baseline (speedup 1.0000x reference)
import functools

import jax
import jax.numpy as jnp
from jax import lax
from jax.experimental import pallas as pl
from jax.experimental.pallas import tpu as pltpu
from jax.experimental.pallas import tpu_sc as plsc

B = 16384
D = 64
NC = 2
NS = 16
NW = NC * NS
BPW = B // NW
CHT = 128
NCHUNK = BPW // CHT
NG = CHT // 16

_FBUF = pltpu.VMEM((D, CHT), jnp.float32)


@functools.partial(
    pl.kernel,
    mesh=plsc.VectorSubcoreMesh(core_axis_name="c", subcore_axis_name="s"),
    compiler_params=pltpu.CompilerParams(needs_layout_passes=False,
                                         use_tc_tiling_on_sc=False),
    out_type=jax.ShapeDtypeStruct((B,), jnp.float32),
    scratch_types=[
        pltpu.VMEM((BPW,), jnp.int32),
        pltpu.VMEM((BPW,), jnp.int32),
        pltpu.VMEM((BPW,), jnp.int32),
        _FBUF, _FBUF, _FBUF, _FBUF,
        pltpu.VMEM((CHT, 2 * D), jnp.float32),
        _FBUF, _FBUF, _FBUF, _FBUF,
        pltpu.VMEM((CHT, 2 * D), jnp.float32),
        pltpu.VMEM((BPW,), jnp.float32),
        pltpu.SemaphoreType.DMA,
        pltpu.SemaphoreType.DMA,
    ],
)
def _sc_score(heads, rels, tails, ereT, eimT, relcat, out,
              hidx, ridx, tidx,
              hreA, himA, treA, timA, relA,
              hreB, himB, treB, timB, relB,
              outv, semA, semB):
    wid = lax.axis_index("s") * NC + lax.axis_index("c")
    base = wid * BPW
    pltpu.sync_copy(heads.at[pl.ds(base, BPW)], hidx)
    pltpu.sync_copy(rels.at[pl.ds(base, BPW)], ridx)
    pltpu.sync_copy(tails.at[pl.ds(base, BPW)], tidx)
    iota16 = lax.iota(jnp.int32, 16)

    def fire(c, bufs, sem):
        hre_x, him_x, tre_x, tim_x, rel_x = bufs
        hsl = hidx.at[pl.ds(c * CHT, CHT)]
        tsl = tidx.at[pl.ds(c * CHT, CHT)]
        pltpu.async_copy(relcat.at[ridx.at[pl.ds(c * CHT, CHT)]],
                         rel_x, sem)
        for d in range(D):
            pltpu.async_copy(ereT.at[d].at[hsl], hre_x.at[d], sem)
            pltpu.async_copy(eimT.at[d].at[hsl], him_x.at[d], sem)
            pltpu.async_copy(ereT.at[d].at[tsl], tre_x.at[d], sem)
            pltpu.async_copy(eimT.at[d].at[tsl], tim_x.at[d], sem)

    def drain(bufs, sem):
        hre_x, him_x, tre_x, tim_x, rel_x = bufs
        for buf in (hre_x, him_x, tre_x, tim_x):
            pltpu.make_async_copy(ereT.at[:, pl.ds(0, CHT)], buf,
                                  sem).wait()
        pltpu.make_async_copy(relcat.at[pl.ds(0, CHT)], rel_x, sem).wait()

    def compute(c, bufs):
        hre_x, him_x, tre_x, tim_x, rel_x = bufs

        def group(g, carry):
            lanes = g * 16 + iota16
            gof = g * 16
            acc = jnp.zeros((16,), jnp.float32)
            col = jnp.zeros((16,), jnp.int32)
            one = jnp.ones((16,), jnp.int32)
            for d in range(D):
                h_re = hre_x[d, pl.ds(gof, 16)]
                h_im = him_x[d, pl.ds(gof, 16)]
                t_re = tre_x[d, pl.ds(gof, 16)]
                t_im = tim_x[d, pl.ds(gof, 16)]
                r_re = plsc.load_gather(rel_x, [lanes, col])
                r_im = plsc.load_gather(rel_x, [lanes, col + D])
                acc = (acc
                       + (h_re * r_re - h_im * r_im) * t_re
                       + (h_re * r_im + h_im * r_re) * t_im)
                col = col + one
            outv[pl.ds(c * CHT + gof, 16)] = acc
            return carry

        lax.fori_loop(0, NG, group, 0)

    bufsA = (hreA, himA, treA, timA, relA)
    bufsB = (hreB, himB, treB, timB, relB)
    fire(0, bufsA, semA)
    fire(1, bufsB, semB)

    def body(i, carry):
        c0 = 2 * i
        drain(bufsA, semA)
        compute(c0, bufsA)
        fire(c0 + 2, bufsA, semA)
        drain(bufsB, semB)
        compute(c0 + 1, bufsB)
        fire(c0 + 3, bufsB, semB)
        return carry

    lax.fori_loop(0, NCHUNK // 2 - 1, body, 0)
    drain(bufsA, semA)
    compute(NCHUNK - 2, bufsA)
    drain(bufsB, semB)
    compute(NCHUNK - 1, bufsB)
    pltpu.sync_copy(outv, out.at[pl.ds(base, BPW)])


def kernel(heads, relations, tails, entity_re, entity_im, relation_re,
           relation_im):
    ereT = entity_re.T
    eimT = entity_im.T
    relcat = jnp.concatenate([relation_re, relation_im], axis=1)
    return _sc_score(heads.astype(jnp.int32), relations.astype(jnp.int32),
                     tails.astype(jnp.int32), ereT, eimT, relcat)


# speedup vs baseline: 8.5471x; 8.5471x over previous
import functools

import jax
import jax.numpy as jnp
from jax import lax
from jax.experimental import pallas as pl
from jax.experimental.pallas import tpu as pltpu
from jax.experimental.pallas import tpu_sc as plsc

B = 16384
D = 64
NC = 2
NS = 16
NW = NC * NS
BPW = B // NW
CHT = 64
NCHUNK = BPW // CHT
NG = CHT // 16
EROWS = 1000000 // 2

_RBUF = pltpu.VMEM((CHT, 128), jnp.float32)
_IBUF = pltpu.VMEM((CHT,), jnp.int32)


@functools.partial(
    pl.kernel,
    mesh=plsc.VectorSubcoreMesh(core_axis_name="c", subcore_axis_name="s"),
    compiler_params=pltpu.CompilerParams(needs_layout_passes=False,
                                         use_tc_tiling_on_sc=True),
    out_type=jax.ShapeDtypeStruct((B,), jnp.float32),
    scratch_types=[
        pltpu.VMEM((BPW,), jnp.int32),
        pltpu.VMEM((BPW,), jnp.int32),
        pltpu.VMEM((BPW,), jnp.int32),
        _IBUF, _IBUF,
        _RBUF, _RBUF, _RBUF, _RBUF,
        pltpu.VMEM((CHT, 2 * D), jnp.float32),
        _IBUF, _IBUF,
        _RBUF, _RBUF, _RBUF, _RBUF,
        pltpu.VMEM((CHT, 2 * D), jnp.float32),
        pltpu.VMEM((BPW,), jnp.float32),
        pltpu.SemaphoreType.DMA,
        pltpu.SemaphoreType.DMA,
    ],
)
def _sc_score(heads, rels, tails, ere2, eim2, relcat, out,
              hidx, ridx, tidx,
              hiA, tiA, hreA, himA, treA, timA, relA,
              hiB, tiB, hreB, himB, treB, timB, relB,
              outv, semA, semB):
    wid = lax.axis_index("s") * NC + lax.axis_index("c")
    base = wid * BPW
    pltpu.sync_copy(heads.at[pl.ds(base, BPW)], hidx)
    pltpu.sync_copy(rels.at[pl.ds(base, BPW)], ridx)
    pltpu.sync_copy(tails.at[pl.ds(base, BPW)], tidx)
    iota16 = lax.iota(jnp.int32, 16)

    def fire(c, bufs, sem):
        hi_x, ti_x, hre_x, him_x, tre_x, tim_x, rel_x = bufs
        for g in range(NG):
            hv = hidx[pl.ds(c * CHT + g * 16, 16)]
            tv = tidx[pl.ds(c * CHT + g * 16, 16)]
            hi_x[pl.ds(g * 16, 16)] = lax.shift_right_logical(hv, 1)
            ti_x[pl.ds(g * 16, 16)] = lax.shift_right_logical(tv, 1)
        pltpu.async_copy(relcat.at[ridx.at[pl.ds(c * CHT, CHT)]],
                         rel_x, sem)
        pltpu.async_copy(ere2.at[hi_x], hre_x, sem)
        pltpu.async_copy(eim2.at[hi_x], him_x, sem)
        pltpu.async_copy(ere2.at[ti_x], tre_x, sem)
        pltpu.async_copy(eim2.at[ti_x], tim_x, sem)

    def drain(bufs, sem):
        hi_x, ti_x, hre_x, him_x, tre_x, tim_x, rel_x = bufs
        for buf in (hre_x, him_x, tre_x, tim_x):
            pltpu.make_async_copy(ere2.at[pl.ds(0, CHT)], buf, sem).wait()
        pltpu.make_async_copy(relcat.at[pl.ds(0, CHT)], rel_x, sem).wait()

    def compute(c, bufs):
        hi_x, ti_x, hre_x, him_x, tre_x, tim_x, rel_x = bufs

        def group(g, carry):
            gof = g * 16
            lanes = gof + iota16
            hv = hidx[pl.ds(c * CHT + gof, 16)]
            tv = tidx[pl.ds(c * CHT + gof, 16)]
            hb = lax.shift_left(lax.bitwise_and(hv, 1), 6)
            tb = lax.shift_left(lax.bitwise_and(tv, 1), 6)
            acc = jnp.zeros((16,), jnp.float32)
            col = jnp.zeros((16,), jnp.int32)
            one = jnp.ones((16,), jnp.int32)
            for d in range(D):
                h_re = plsc.load_gather(hre_x, [lanes, hb + col])
                h_im = plsc.load_gather(him_x, [lanes, hb + col])
                t_re = plsc.load_gather(tre_x, [lanes, tb + col])
                t_im = plsc.load_gather(tim_x, [lanes, tb + col])
                r_re = plsc.load_gather(rel_x, [lanes, col])
                r_im = plsc.load_gather(rel_x, [lanes, col + D])
                acc = (acc
                       + (h_re * r_re - h_im * r_im) * t_re
                       + (h_re * r_im + h_im * r_re) * t_im)
                col = col + one
            outv[pl.ds(c * CHT + gof, 16)] = acc
            return carry

        lax.fori_loop(0, NG, group, 0)

    bufsA = (hiA, tiA, hreA, himA, treA, timA, relA)
    bufsB = (hiB, tiB, hreB, himB, treB, timB, relB)
    fire(0, bufsA, semA)
    fire(1, bufsB, semB)

    def body(i, carry):
        c0 = 2 * i
        drain(bufsA, semA)
        compute(c0, bufsA)
        fire(c0 + 2, bufsA, semA)
        drain(bufsB, semB)
        compute(c0 + 1, bufsB)
        fire(c0 + 3, bufsB, semB)
        return carry

    lax.fori_loop(0, NCHUNK // 2 - 1, body, 0)
    drain(bufsA, semA)
    compute(NCHUNK - 2, bufsA)
    drain(bufsB, semB)
    compute(NCHUNK - 1, bufsB)
    pltpu.sync_copy(outv, out.at[pl.ds(base, BPW)])


def kernel(heads, relations, tails, entity_re, entity_im, relation_re,
           relation_im):
    ere2 = entity_re.reshape(EROWS, 2 * D)
    eim2 = entity_im.reshape(EROWS, 2 * D)
    relcat = jnp.concatenate([relation_re, relation_im], axis=1)
    return _sc_score(heads.astype(jnp.int32), relations.astype(jnp.int32),
                     tails.astype(jnp.int32), ere2, eim2, relcat)


# speedup vs baseline: 17.9605x; 2.1014x over previous
import functools

import jax
import jax.numpy as jnp
from jax import lax
from jax.experimental import pallas as pl
from jax.experimental.pallas import tpu as pltpu
from jax.experimental.pallas import tpu_sc as plsc

B = 16384
D = 64
NC = 2
NS = 16
NW = NC * NS
BPW = B // NW
CHT = 16
NCHUNK = BPW // CHT
ETILES = 1000000 // 8

_ROWBUF = pltpu.VMEM((2, 8, D), jnp.float32)


@functools.partial(
    pl.kernel,
    mesh=plsc.VectorSubcoreMesh(core_axis_name="c", subcore_axis_name="s"),
    compiler_params=pltpu.CompilerParams(needs_layout_passes=False,
                                         use_tc_tiling_on_sc=True),
    out_type=jax.ShapeDtypeStruct((B,), jnp.float32),
    scratch_types=[
        pltpu.VMEM((BPW,), jnp.int32),
        pltpu.VMEM((BPW,), jnp.int32),
        pltpu.VMEM((BPW,), jnp.int32),
        _ROWBUF, _ROWBUF, _ROWBUF, _ROWBUF,
        pltpu.VMEM((CHT, 2 * D), jnp.float32),
        _ROWBUF, _ROWBUF, _ROWBUF, _ROWBUF,
        pltpu.VMEM((CHT, 2 * D), jnp.float32),
        pltpu.VMEM((BPW,), jnp.float32),
        pltpu.SemaphoreType.DMA,
        pltpu.SemaphoreType.DMA,
    ],
)
def _sc_score(heads, rels, tails, ere3, eim3, relcat, out,
              hidx, ridx, tidx,
              hreA, himA, treA, timA, relA,
              hreB, himB, treB, timB, relB,
              outv, semA, semB):
    wid = lax.axis_index("s") * NC + lax.axis_index("c")
    base = wid * BPW
    pltpu.sync_copy(heads.at[pl.ds(base, BPW)], hidx)
    pltpu.sync_copy(rels.at[pl.ds(base, BPW)], ridx)
    pltpu.sync_copy(tails.at[pl.ds(base, BPW)], tidx)
    lanes = lax.iota(jnp.int32, CHT)
    lhi = lax.shift_right_logical(lanes, 3)
    llo = lax.bitwise_and(lanes, 7)

    def fire(c, bufs, sem):
        hre_x, him_x, tre_x, tim_x, rel_x = bufs
        hv = hidx[pl.ds(c * CHT, CHT)]
        tv = tidx[pl.ds(c * CHT, CHT)]
        ht = lax.shift_right_logical(hv, 3)
        tt = lax.shift_right_logical(tv, 3)
        hs = lax.bitwise_and(hv, 7)
        ts = lax.bitwise_and(tv, 7)
        pltpu.async_copy(relcat.at[ridx.at[pl.ds(c * CHT, CHT)]],
                         rel_x, sem)
        for i in range(CHT):
            dst = (i // 8, i % 8)
            pltpu.async_copy(ere3.at[ht[i], hs[i]], hre_x.at[dst], sem)
            pltpu.async_copy(eim3.at[ht[i], hs[i]], him_x.at[dst], sem)
            pltpu.async_copy(ere3.at[tt[i], ts[i]], tre_x.at[dst], sem)
            pltpu.async_copy(eim3.at[tt[i], ts[i]], tim_x.at[dst], sem)

    def drain(bufs, sem):
        hre_x, him_x, tre_x, tim_x, rel_x = bufs
        for buf in (hre_x, him_x, tre_x, tim_x):
            for i in range(CHT):
                pltpu.make_async_copy(ere3.at[0, 0],
                                      buf.at[i // 8, i % 8], sem).wait()
        pltpu.make_async_copy(relcat.at[pl.ds(0, CHT)], rel_x, sem).wait()

    def compute(c, bufs):
        hre_x, him_x, tre_x, tim_x, rel_x = bufs
        acc = jnp.zeros((CHT,), jnp.float32)
        col = jnp.zeros((CHT,), jnp.int32)
        one = jnp.ones((CHT,), jnp.int32)
        for d in range(D):
            h_re = plsc.load_gather(hre_x, [lhi, llo, col])
            h_im = plsc.load_gather(him_x, [lhi, llo, col])
            t_re = plsc.load_gather(tre_x, [lhi, llo, col])
            t_im = plsc.load_gather(tim_x, [lhi, llo, col])
            r_re = plsc.load_gather(rel_x, [lanes, col])
            r_im = plsc.load_gather(rel_x, [lanes, col + D])
            acc = (acc
                   + (h_re * r_re - h_im * r_im) * t_re
                   + (h_re * r_im + h_im * r_re) * t_im)
            col = col + one
        outv[pl.ds(c * CHT, CHT)] = acc

    bufsA = (hreA, himA, treA, timA, relA)
    bufsB = (hreB, himB, treB, timB, relB)
    fire(0, bufsA, semA)
    fire(1, bufsB, semB)

    def body(i, carry):
        c0 = 2 * i
        drain(bufsA, semA)
        compute(c0, bufsA)
        fire(c0 + 2, bufsA, semA)
        drain(bufsB, semB)
        compute(c0 + 1, bufsB)
        fire(c0 + 3, bufsB, semB)
        return carry

    lax.fori_loop(0, NCHUNK // 2 - 1, body, 0)
    drain(bufsA, semA)
    compute(NCHUNK - 2, bufsA)
    drain(bufsB, semB)
    compute(NCHUNK - 1, bufsB)
    pltpu.sync_copy(outv, out.at[pl.ds(base, BPW)])


def kernel(heads, relations, tails, entity_re, entity_im, relation_re,
           relation_im):
    ere3 = entity_re.reshape(ETILES, 8, D)
    eim3 = entity_im.reshape(ETILES, 8, D)
    relcat = jnp.concatenate([relation_re, relation_im], axis=1)
    return _sc_score(heads.astype(jnp.int32), relations.astype(jnp.int32),
                     tails.astype(jnp.int32), ere3, eim3, relcat)
